# baseline (speedup 1.0000x reference)
import functools

import jax
import jax.numpy as jnp
from jax.experimental import pallas as pl
from jax.experimental.pallas import tpu as pltpu

SEQ = 8192
D = 768
H = 24576
K = 150

BT = 256
BH = 2048


def _enc_body(x_ref, w_ref, b_ref, out_ref):
    out_ref[...] = jax.lax.dot_general(
        x_ref[...], w_ref[...], (((1,), (1,)), ((), ())),
        preferred_element_type=jnp.float32) + b_ref[...][None, :]


def _encoder(x, W_enc, b_enc):
    grid = (H // BH, SEQ // BT)
    return pl.pallas_call(
        _enc_body,
        grid=grid,
        in_specs=[
            pl.BlockSpec((BT, D), lambda h, t: (t, 0)),
            pl.BlockSpec((BH, D), lambda h, t: (h, 0)),
            pl.BlockSpec((BH,), lambda h, t: (h,)),
        ],
        out_specs=pl.BlockSpec((BT, BH), lambda h, t: (t, h)),
        out_shape=jax.ShapeDtypeStruct((SEQ, H), jnp.float32),
    )(x, W_enc, b_enc)


def _dec_body(s_ref, w_ref, b_ref, out_ref):
    k = pl.program_id(1)

    @pl.when(k == 0)
    def _init():
        out_ref[...] = jnp.broadcast_to(b_ref[...][None, :], out_ref.shape)

    out_ref[...] += jax.lax.dot_general(
        s_ref[...], w_ref[...], (((1,), (1,)), ((), ())),
        preferred_element_type=jnp.float32)


def _decoder(sae, W_dec, b_dec):
    grid = (SEQ // BT, H // BH)
    return pl.pallas_call(
        _dec_body,
        grid=grid,
        in_specs=[
            pl.BlockSpec((BT, BH), lambda t, k: (t, k)),
            pl.BlockSpec((D, BH), lambda t, k: (0, k)),
            pl.BlockSpec((D,), lambda t, k: (0,)),
        ],
        out_specs=pl.BlockSpec((BT, D), lambda t, k: (t, 0)),
        out_shape=jax.ShapeDtypeStruct((SEQ, D), jnp.float32),
    )(sae, W_dec, b_dec)


def kernel(llm_activations, W_enc, b_enc, W_dec, b_dec):
    x = llm_activations.reshape(SEQ, D)
    pre = _encoder(x, W_enc, b_enc)
    vals, idx = jax.lax.top_k(pre, K)
    z = jnp.zeros_like(pre)
    s = jnp.arange(SEQ)[:, None]
    sae = z.at[s, idx].set(vals)
    out = _decoder(sae, W_dec, b_dec)
    return out.reshape(1, SEQ, D)


# speedup vs baseline: 27.4335x; 27.4335x over previous
import functools

import jax
import jax.numpy as jnp
from jax.experimental import pallas as pl
from jax.experimental.pallas import tpu as pltpu

SEQ = 8192
D = 768
H = 24576
K = 150

BT = 256
BH = 2048


def _enc_body(x_ref, w_ref, b_ref, out_ref):
    out_ref[...] = jax.lax.dot_general(
        x_ref[...], w_ref[...], (((1,), (1,)), ((), ())),
        preferred_element_type=jnp.float32) + b_ref[...][None, :]


def _encoder(x, W_enc, b_enc):
    grid = (H // BH, SEQ // BT)
    return pl.pallas_call(
        _enc_body,
        grid=grid,
        in_specs=[
            pl.BlockSpec((BT, D), lambda h, t: (t, 0)),
            pl.BlockSpec((BH, D), lambda h, t: (h, 0)),
            pl.BlockSpec((BH,), lambda h, t: (h,)),
        ],
        out_specs=pl.BlockSpec((BT, BH), lambda h, t: (t, h)),
        out_shape=jax.ShapeDtypeStruct((SEQ, H), jnp.float32),
    )(x, W_enc, b_enc)


BTS = 128
SEL_ITERS = 26


def _sel_body(pre_ref, t_ref):
    x = pre_ref[...]
    lo = jnp.min(x, axis=1) - 1.0
    hi = jnp.max(x, axis=1)

    def it(_, c):
        lo, hi = c
        mid = 0.5 * (lo + hi)
        cnt = jnp.sum(jnp.where(x > mid[:, None], 1.0, 0.0), axis=1)
        pred = cnt >= K
        return (jnp.where(pred, mid, lo), jnp.where(pred, hi, mid))

    lo, hi = jax.lax.fori_loop(0, SEL_ITERS, it, (lo, hi))
    t_ref[...] = lo


def _select_threshold(pre):
    return pl.pallas_call(
        _sel_body,
        grid=(SEQ // BTS,),
        in_specs=[pl.BlockSpec((BTS, H), lambda t: (t, 0))],
        out_specs=pl.BlockSpec((BTS,), lambda t: (t,)),
        out_shape=jax.ShapeDtypeStruct((SEQ,), jnp.float32),
    )(pre)


def _dec_body(p_ref, t_ref, w_ref, b_ref, out_ref):
    k = pl.program_id(1)

    @pl.when(k == 0)
    def _init():
        out_ref[...] = jnp.broadcast_to(b_ref[...][None, :], out_ref.shape)

    p = p_ref[...]
    s = jnp.where(p > t_ref[...][:, None], p, 0.0)
    out_ref[...] += jax.lax.dot_general(
        s, w_ref[...], (((1,), (1,)), ((), ())),
        preferred_element_type=jnp.float32)


def _decoder(pre, thr, W_dec, b_dec):
    grid = (SEQ // BT, H // BH)
    return pl.pallas_call(
        _dec_body,
        grid=grid,
        in_specs=[
            pl.BlockSpec((BT, BH), lambda t, k: (t, k)),
            pl.BlockSpec((BT,), lambda t, k: (t,)),
            pl.BlockSpec((D, BH), lambda t, k: (0, k)),
            pl.BlockSpec((D,), lambda t, k: (0,)),
        ],
        out_specs=pl.BlockSpec((BT, D), lambda t, k: (t, 0)),
        out_shape=jax.ShapeDtypeStruct((SEQ, D), jnp.float32),
    )(pre, thr, W_dec, b_dec)


def kernel(llm_activations, W_enc, b_enc, W_dec, b_dec):
    x = llm_activations.reshape(SEQ, D)
    pre = _encoder(x, W_enc, b_enc)
    thr = _select_threshold(pre)
    out = _decoder(pre, thr, W_dec, b_dec)
    return out.reshape(1, SEQ, D)
